# baseline (speedup 1.0000x reference)
import functools

import jax
import jax.numpy as jnp
from jax import lax
from jax.experimental import pallas as pl
from jax.experimental.pallas import tpu as pltpu
from jax.experimental.pallas import tpu_sc as plsc

_R = 8
_NBUF = 4
_LANES = 16


def _make_sc_kernel(rows, D, rows_per_worker, S):
    nchunk = rows_per_worker // _R
    ngroup = nchunk // _NBUF
    mesh = plsc.VectorSubcoreMesh(core_axis_name="c", subcore_axis_name="s")

    @functools.partial(
        pl.kernel,
        mesh=mesh,
        out_type=jax.ShapeDtypeStruct((rows, D), jnp.float32),
        scratch_types=[
            pltpu.VMEM((_NBUF, _R, D), jnp.float32),
            pltpu.VMEM((_NBUF, _R, D), jnp.float32),
            pltpu.VMEM((_NBUF, _R, D), jnp.float32),
            pltpu.SemaphoreType.DMA((_NBUF,)),
            pltpu.SemaphoreType.DMA((_NBUF,)),
            pltpu.SemaphoreType.DMA((_NBUF,)),
        ],
    )
    def sc_kernel(x_hbm, pt_hbm, out_hbm, xb, pb, ob, xs, ps, os):
        wid = lax.axis_index("s") * 2 + lax.axis_index("c")
        base = wid * rows_per_worker
        pbase = (base % S)

        def start_in(slot, c):
            row = base + c * _R
            prow = pbase + c * _R
            pltpu.async_copy(x_hbm.at[pl.ds(row, _R), :], xb.at[slot], xs.at[slot])
            pltpu.async_copy(pt_hbm.at[pl.ds(prow, _R), :], pb.at[slot], ps.at[slot])

        def wait_in(slot, c):
            row = base + c * _R
            prow = pbase + c * _R
            pltpu.make_async_copy(x_hbm.at[pl.ds(row, _R), :], xb.at[slot], xs.at[slot]).wait()
            pltpu.make_async_copy(pt_hbm.at[pl.ds(prow, _R), :], pb.at[slot], ps.at[slot]).wait()

        def start_out(slot, c):
            row = base + c * _R
            pltpu.async_copy(ob.at[slot], out_hbm.at[pl.ds(row, _R), :], os.at[slot])

        def wait_out(slot, c):
            row = base + c * _R
            pltpu.make_async_copy(ob.at[slot], out_hbm.at[pl.ds(row, _R), :], os.at[slot]).wait()

        for b in range(_NBUF):
            start_in(b, b)

        def group(g, _):
            for b in range(_NBUF):
                c = g * _NBUF + b
                wait_in(b, c)

                @pl.when(c >= _NBUF)
                def _():
                    wait_out(b, c - _NBUF)

                def row_add(r, _):
                    for j in range(D // _LANES):
                        sl = pl.ds(j * _LANES, _LANES)
                        ob[b, r, sl] = xb[b, r, sl] + pb[b, r, sl]
                    return 0

                lax.fori_loop(0, _R, row_add, 0, unroll=False)
                start_out(b, c)

                @pl.when(c + _NBUF < nchunk)
                def _():
                    start_in(b, c + _NBUF)
            return 0

        lax.fori_loop(0, ngroup, group, 0, unroll=False)

        for b in range(_NBUF):
            wait_out(b, nchunk - _NBUF + b)

    return sc_kernel


def kernel(x, pos_table):
    B, S, D = x.shape
    rows = B * S
    rows_per_worker = rows // 32
    xf = x.reshape(rows, D)
    out = _make_sc_kernel(rows, D, rows_per_worker, S)(xf, pos_table)
    return out.reshape(B, S, D)


# speedup vs baseline: 1.2237x; 1.2237x over previous
import functools

import jax
import jax.numpy as jnp
from jax import lax
from jax.experimental import pallas as pl
from jax.experimental.pallas import tpu as pltpu
from jax.experimental.pallas import tpu_sc as plsc

_R = 4
_NBUF = 4
_LANES = 16
_NW = 32


def _make_sc_kernel(B, S, D):
    s_per_worker = S // _NW
    nchunk = s_per_worker // _R
    ngroup = nchunk // _NBUF
    mesh = plsc.VectorSubcoreMesh(core_axis_name="c", subcore_axis_name="s")

    @functools.partial(
        pl.kernel,
        mesh=mesh,
        out_type=jax.ShapeDtypeStruct((B, S, D), jnp.float32),
        scratch_types=[
            pltpu.VMEM((_NBUF, B, _R, D), jnp.float32),
            pltpu.VMEM((_NBUF, _R, D), jnp.float32),
            pltpu.SemaphoreType.DMA((_NBUF,)),
            pltpu.SemaphoreType.DMA((_NBUF,)),
            pltpu.SemaphoreType.DMA((_NBUF,)),
        ],
    )
    def sc_kernel(x_hbm, pt_hbm, out_hbm, xb, pb, xs, ps, os):
        wid = lax.axis_index("s") * 2 + lax.axis_index("c")
        base = wid * s_per_worker

        def start_in(slot, c):
            row = base + c * _R
            for bb in range(B):
                pltpu.async_copy(
                    x_hbm.at[bb, pl.ds(row, _R), :], xb.at[slot, bb], xs.at[slot])
            pltpu.async_copy(pt_hbm.at[pl.ds(row, _R), :], pb.at[slot], ps.at[slot])

        def wait_in(slot, c):
            row = base + c * _R
            for bb in range(B):
                pltpu.make_async_copy(
                    x_hbm.at[bb, pl.ds(row, _R), :], xb.at[slot, bb], xs.at[slot]).wait()
            pltpu.make_async_copy(
                pt_hbm.at[pl.ds(row, _R), :], pb.at[slot], ps.at[slot]).wait()

        def start_out(slot, c):
            row = base + c * _R
            for bb in range(B):
                pltpu.async_copy(
                    xb.at[slot, bb], out_hbm.at[bb, pl.ds(row, _R), :], os.at[slot])

        def wait_out(slot, c):
            row = base + c * _R
            for bb in range(B):
                pltpu.make_async_copy(
                    xb.at[slot, bb], out_hbm.at[bb, pl.ds(row, _R), :], os.at[slot]).wait()

        for b in range(_NBUF):
            start_in(b, b)

        def group(g, _):
            for b in range(_NBUF):
                c = g * _NBUF + b
                wait_in(b, c)

                @pl.when(c >= _NBUF)
                def _():
                    wait_out(b, c - _NBUF)

                def row_add(r, _):
                    for j in range(D // _LANES):
                        sl = pl.ds(j * _LANES, _LANES)
                        pval = pb[b, r, sl]
                        for bb in range(B):
                            plsc.addupdate(xb.at[b, bb, r, sl], pval)
                    return 0

                lax.fori_loop(0, _R, row_add, 0, unroll=False)
                start_out(b, c)

                @pl.when(c + _NBUF < nchunk)
                def _():
                    start_in(b, c + _NBUF)
            return 0

        lax.fori_loop(0, ngroup, group, 0, unroll=False)

        for b in range(_NBUF):
            wait_out(b, nchunk - _NBUF + b)

    return sc_kernel


def kernel(x, pos_table):
    B, S, D = x.shape
    return _make_sc_kernel(B, S, D)(x, pos_table)


# speedup vs baseline: 1.2613x; 1.0307x over previous
import functools

import jax
import jax.numpy as jnp
from jax import lax
from jax.experimental import pallas as pl
from jax.experimental.pallas import tpu as pltpu
from jax.experimental.pallas import tpu_sc as plsc

_R = 8
_NBUF = 2
_LANES = 16
_NW = 32


def _make_sc_kernel(B, S, D):
    s_per_worker = S // _NW
    nchunk = s_per_worker // _R
    ngroup = nchunk // _NBUF
    mesh = plsc.VectorSubcoreMesh(core_axis_name="c", subcore_axis_name="s")

    @functools.partial(
        pl.kernel,
        mesh=mesh,
        out_type=jax.ShapeDtypeStruct((B, S, D), jnp.float32),
        scratch_types=[
            pltpu.VMEM((_NBUF, B, _R, D), jnp.float32),
            pltpu.VMEM((_NBUF, _R, D), jnp.float32),
            pltpu.SemaphoreType.DMA((_NBUF,)),
            pltpu.SemaphoreType.DMA((_NBUF,)),
            pltpu.SemaphoreType.DMA((_NBUF,)),
        ],
    )
    def sc_kernel(x_hbm, pt_hbm, out_hbm, xb, pb, xs, ps, os):
        wid = lax.axis_index("s") * 2 + lax.axis_index("c")
        base = wid * s_per_worker

        def start_in(slot, c):
            row = base + c * _R
            for bb in range(B):
                pltpu.async_copy(
                    x_hbm.at[bb, pl.ds(row, _R), :], xb.at[slot, bb], xs.at[slot])
            pltpu.async_copy(pt_hbm.at[pl.ds(row, _R), :], pb.at[slot], ps.at[slot])

        def wait_in(slot, c):
            row = base + c * _R
            for bb in range(B):
                pltpu.make_async_copy(
                    x_hbm.at[bb, pl.ds(row, _R), :], xb.at[slot, bb], xs.at[slot]).wait()
            pltpu.make_async_copy(
                pt_hbm.at[pl.ds(row, _R), :], pb.at[slot], ps.at[slot]).wait()

        def start_out(slot, c):
            row = base + c * _R
            for bb in range(B):
                pltpu.async_copy(
                    xb.at[slot, bb], out_hbm.at[bb, pl.ds(row, _R), :], os.at[slot])

        def wait_out(slot, c):
            row = base + c * _R
            for bb in range(B):
                pltpu.make_async_copy(
                    xb.at[slot, bb], out_hbm.at[bb, pl.ds(row, _R), :], os.at[slot]).wait()

        for b in range(_NBUF):
            start_in(b, b)

        def group(g, _):
            for b in range(_NBUF):
                c = g * _NBUF + b
                wait_in(b, c)

                @pl.when(c >= _NBUF)
                def _():
                    wait_out(b, c - _NBUF)

                def row_add(r, _):
                    for j in range(D // _LANES):
                        sl = pl.ds(j * _LANES, _LANES)
                        pval = pb[b, r, sl]
                        for bb in range(B):
                            plsc.addupdate(xb.at[b, bb, r, sl], pval)
                    return 0

                lax.fori_loop(0, _R, row_add, 0, unroll=False)
                start_out(b, c)

                @pl.when(c + _NBUF < nchunk)
                def _():
                    start_in(b, c + _NBUF)
            return 0

        lax.fori_loop(0, ngroup, group, 0, unroll=False)

        for b in range(_NBUF):
            wait_out(b, nchunk - _NBUF + b)

    return sc_kernel


def kernel(x, pos_table):
    B, S, D = x.shape
    return _make_sc_kernel(B, S, D)(x, pos_table)
